# baseline (speedup 1.0000x reference)
import jax
import jax.numpy as jnp
import numpy as np
from jax.experimental import pallas as pl
from jax.experimental.pallas import tpu as pltpu

B, N, F_IN, D = 4096, 11, 11, 64
BN = B * N
ND = N * D
ND2 = 2 * ND
NF = N * F_IN
NE = N * N
NP = (N + 1) // 2
AW = NP * ND2

TB1 = 256
TB2 = 512
NT1 = B // TB1
NT2 = B // TB2

_F32 = jnp.float32


def _bcast_map():
    m = np.zeros((NE, AW), np.float32)
    for i in range(N):
        for j in range(N):
            c = (i // 2) * ND2 + j * 128 + (i % 2) * D
            m[i * N + j, c:c + D] = 1.0
    return jnp.asarray(m)


def _tree_sum(terms):
    while len(terms) > 1:
        nxt = [terms[k] + terms[k + 1] for k in range(0, len(terms) - 1, 2)]
        if len(terms) % 2:
            nxt.append(terms[-1])
        terms = nxt
    return terms[0]


def _kernel(dat_ref, wbd_ref, b2_ref, embT_ref, aemi_ref, aemj_ref,
            aij_ref, ebi_ref, ebj_ref, gfold_ref, bmap_ref, g_ref, be_ref,
            w1_ref, b1_ref, g1_ref, e1_ref,
            w2_ref, b2f_ref, g2_ref, e2_ref,
            w3_ref, b3_ref, g3_ref, e3_ref,
            w4_ref, b4_ref, g4_ref, e4_ref,
            w5_ref, b5_ref, g5_ref, e5_ref,
            w6_ref, b6_ref, y_ref, h0s, sum_s, sq_s):
    t = pl.program_id(0)

    @pl.when(t < NT1)
    def _gat_phase():
        h2 = jnp.dot(dat_ref[:], wbd_ref[:],
                     preferred_element_type=_F32) + b2_ref[:]
        ci = jnp.dot(aemi_ref[:], embT_ref[:], preferred_element_type=_F32)
        cj = jnp.dot(aemj_ref[:], embT_ref[:], preferred_element_type=_F32)
        cij = (jnp.dot(ci, ebi_ref[:], preferred_element_type=_F32)
               + jnp.dot(cj, ebj_ref[:], preferred_element_type=_F32))
        logits = jnp.dot(h2, aij_ref[:], preferred_element_type=_F32) + cij
        logits = jnp.where(logits >= 0, logits, 0.2 * logits)
        e = jnp.exp(logits)
        s = jnp.dot(e, gfold_ref[:], preferred_element_type=_F32)
        r = 1.0 / (s + 1e-16)
        attn = e * jnp.dot(r, ebi_ref[:], preferred_element_type=_F32)
        a_all = jnp.dot(attn, bmap_ref[:], preferred_element_type=_F32)
        row = t * TB1
        for p in range(NP):
            blk = a_all[:, p * ND2:(p + 1) * ND2] * h2
            res = _tree_sum([blk[:, k * 128:(k + 1) * 128] for k in range(N)])
            if p < NP - 1:
                h0s[pl.ds(row, TB1), p * 128:(p + 1) * 128] = res
            else:
                h0s[pl.ds(row, TB1), p * 128:p * 128 + D] = res[:, 0:D]
        o = h0s[pl.ds(row, TB1), :]
        ones = jnp.ones((1, TB1), dtype=_F32)
        s704 = jnp.dot(ones, o, preferred_element_type=_F32)
        q704 = jnp.dot(ones, o * o, preferred_element_type=_F32)
        s64 = _tree_sum([s704[:, n * D:(n + 1) * D] for n in range(N)])
        q64 = _tree_sum([q704[:, n * D:(n + 1) * D] for n in range(N)])

        @pl.when(t == 0)
        def _init():
            sum_s[:] = s64
            sq_s[:] = q64

        @pl.when(t != 0)
        def _acc():
            sum_s[:] = sum_s[:] + s64
            sq_s[:] = sq_s[:] + q64

    @pl.when(t >= NT1)
    def _fnn_phase():
        mraw = sum_s[:] * (1.0 / BN)
        var = sq_s[:] * (1.0 / BN) - mraw * mraw
        inv = jax.lax.rsqrt(var + 1e-5)
        scale = inv * g_ref[:]
        shift = be_ref[:] - mraw * scale
        scale704 = jnp.concatenate([scale] * N, axis=1)
        shift704 = jnp.concatenate([shift] * N, axis=1)
        x = h0s[pl.ds((t - NT1) * TB2, TB2), :]
        h = x * scale704 + shift704
        h = jnp.where(h >= 0, h, 0.01 * h)

        def _hidden(x, w, b, g, e):
            z = jnp.dot(x, w, preferred_element_type=_F32) + b
            n = z.shape[1]
            v = jnp.sum(z * z, axis=1, keepdims=True) * (1.0 / n)
            r = jax.lax.rsqrt(v + 1e-5)
            return jnp.maximum(z * r * g + e, 0.0)

        h = _hidden(h, w1_ref[:], b1_ref[:], g1_ref[:], e1_ref[:])
        h = _hidden(h, w2_ref[:], b2f_ref[:], g2_ref[:], e2_ref[:])
        h = _hidden(h, w3_ref[:], b3_ref[:], g3_ref[:], e3_ref[:])
        h = _hidden(h, w4_ref[:], b4_ref[:], g4_ref[:], e4_ref[:])
        h = _hidden(h, w5_ref[:], b5_ref[:], g5_ref[:], e5_ref[:])
        y_ref[:] = (jnp.dot(h, w6_ref[:], preferred_element_type=_F32)
                    + b6_ref[:])


def _full(shape):
    return pl.BlockSpec(shape, lambda t: tuple(0 for _ in shape))


@jax.jit
def kernel(data, edge_index, gat_params, bn_params, emb, fnn_params):
    del edge_index
    dat = data.reshape(B, NF)
    eye = jnp.eye(N, dtype=_F32)
    lw = gat_params['lin_W']
    wbd = jnp.kron(eye, jnp.concatenate([lw, lw], axis=1))
    lb2 = jnp.concatenate([gat_params['lin_b']] * 2)
    b2 = jnp.tile(lb2, N).reshape(1, ND2)
    ai = jnp.kron(eye, gat_params['att_i'].reshape(D, 1))
    aj = jnp.kron(eye, gat_params['att_j'].reshape(D, 1))
    ebi = jnp.kron(eye, jnp.ones((1, N), _F32))
    ebj = jnp.tile(eye, (1, N))
    aij = jnp.dot(ai, ebi) + jnp.dot(aj, ebj)
    aij2 = jnp.pad(aij.reshape(N, D, NE),
                   ((0, 0), (0, D), (0, 0))).reshape(ND2, NE)
    gfold = jnp.kron(eye, jnp.ones((N, 1), _F32))
    bmap = _bcast_map()
    aemi = gat_params['att_em_i'].reshape(1, D)
    aemj = gat_params['att_em_j'].reshape(1, D)
    embT = emb.T
    g, be = bn_params

    fnn_flat = []
    fnn_specs = []
    for p in fnn_params:
        w = p[0]
        vs = list(p[1:])
        if len(p) == 4:
            w = w - jnp.mean(w, axis=1, keepdims=True)
            vs[0] = vs[0] - jnp.mean(vs[0])
        fnn_flat.append(w)
        fnn_specs.append(_full(w.shape))
        for v in vs:
            fnn_flat.append(v.reshape(1, -1))
            fnn_specs.append(_full((1, v.shape[0])))

    y = pl.pallas_call(
        _kernel,
        grid=(NT1 + NT2,),
        in_specs=[
            pl.BlockSpec((TB1, NF), lambda t: (jnp.minimum(t, NT1 - 1), 0)),
            _full((NF, ND2)), _full((1, ND2)), _full((D, N)),
            _full((1, D)), _full((1, D)),
            _full((ND2, NE)), _full((N, NE)), _full((N, NE)),
            _full((NE, N)), _full((NE, AW)),
            _full((1, D)), _full((1, D)),
        ] + fnn_specs,
        out_specs=pl.BlockSpec(
            (TB2, 7), lambda t: (jnp.maximum(t - NT1, 0), 0)),
        out_shape=jax.ShapeDtypeStruct((B, 7), _F32),
        scratch_shapes=[pltpu.VMEM((B, ND), _F32),
                        pltpu.VMEM((1, D), _F32),
                        pltpu.VMEM((1, D), _F32)],
        compiler_params=pltpu.CompilerParams(
            dimension_semantics=("arbitrary",)),
    )(dat, wbd, b2, embT, aemi, aemj, aij2, ebi, ebj, gfold, bmap,
      g.reshape(1, D), be.reshape(1, D), *fnn_flat)
    return y
